# baseline (speedup 1.0000x reference)
import functools

import jax
import jax.numpy as jnp
from jax import lax
from jax.experimental import pallas as pl
from jax.experimental.pallas import tpu as pltpu
from jax.experimental.pallas import tpu_sc as plsc

VOCAB_DIM = 8192
TOKENS = 4 * 2048
NC, NS = 2, 16
NW = NC * NS
TPW = TOKENS // NW
K = 2
CHUNKS = TPW // K
NB = 7
G = 2
DRAIN = NB - G
SLOTS = CHUNKS + DRAIN


def _body(idx_hbm, table_hbm, out_hbm, idx_v, *bufsems):
    bufs, sgs, sws = bufsems[:NB], bufsems[NB:2 * NB], bufsems[2 * NB:]
    wid = lax.axis_index("s") * NC + lax.axis_index("c")
    pltpu.sync_copy(idx_hbm.at[pl.ds(wid * CHUNKS, CHUNKS)], idx_v)
    base = wid * TPW

    for j in range(G):
        pltpu.async_copy(table_hbm.at[idx_v.at[j]], bufs[j], sgs[j])

    def step(g, carry):
        for b in range(NB):
            j = NB * g + b
            q = (b + G) % NB

            @pl.when((j >= DRAIN) & (j < CHUNKS + DRAIN))
            def _drain():
                pltpu.make_async_copy(
                    bufs[q], out_hbm.at[pl.ds(base, K)], sws[q]).wait()

            @pl.when(j + G < CHUNKS)
            def _prefetch():
                pltpu.async_copy(
                    table_hbm.at[idx_v.at[j + G]], bufs[q], sgs[q])

            @pl.when(j < CHUNKS)
            def _emit():
                pltpu.make_async_copy(
                    table_hbm.at[idx_v.at[j]], bufs[b], sgs[b]).wait()
                pltpu.async_copy(
                    bufs[b], out_hbm.at[pl.ds(base + j * K, K)], sws[b])
        return carry

    lax.fori_loop(0, (SLOTS + NB - 1) // NB, step, 0, unroll=False)


_mesh = plsc.VectorSubcoreMesh(core_axis_name="c", subcore_axis_name="s")

_gather = functools.partial(
    pl.kernel,
    mesh=_mesh,
    out_type=jax.ShapeDtypeStruct((TOKENS, VOCAB_DIM), jnp.float32),
    scratch_types=(
        [pltpu.VMEM((CHUNKS, K), jnp.int32)]
        + [pltpu.VMEM((K, VOCAB_DIM), jnp.float32)] * NB
        + [pltpu.SemaphoreType.DMA] * (2 * NB)
    ),
)(_body)


def kernel(x, vocab_table):
    idx = x.reshape(TOKENS).astype(jnp.int32).reshape(NW * CHUNKS, K)
    out = _gather(idx, vocab_table)
    return out.reshape(x.shape + (VOCAB_DIM,))
